# baseline (speedup 1.0000x reference)
import functools

import jax
import jax.numpy as jnp
from jax import lax
from jax.experimental import pallas as pl
from jax.experimental.pallas import tpu as pltpu

N_PART = 13
NP = 16
S_DIM = 3
HID = 64
N_LAYERS = 5
BS = 64


def _lnk(x, g, b):
    m = jnp.mean(x, axis=-1, keepdims=True)
    v = jnp.mean((x - m) ** 2, axis=-1, keepdims=True)
    return (x - m) / jnp.sqrt(v + 1e-5) * g + b


def _silu(x):
    return x * jax.nn.sigmoid(x)


def _egnn_kernel(coord_ref, h0_ref, eW1ab_ref, wrwe_ref,
                 eb1_ref, eg1_ref, ebe1_ref, eW2_ref, eb2_ref, eg2_ref,
                 ebe2_ref, aw_ref, ab_ref, cW1_ref, cb1_ref, cg1_ref,
                 cbe1_ref, cw2_ref, nW1_ref, nb1_ref, ng1_ref,
                 nbe1_ref, nW2_ref, nb2_ref, out_ref):
    bs = coord_ref.shape[0]
    NI = N_PART
    R = bs * NI * NP
    f32 = jnp.float32

    coord = coord_ref[...]
    x_init = coord
    h = jnp.broadcast_to(h0_ref[...], (bs * NP, HID))

    ii = lax.broadcasted_iota(jnp.int32, (1, NI, NP, 1), 1)
    jj = lax.broadcasted_iota(jnp.int32, (1, NI, NP, 1), 2)
    mask4 = ((ii != jj) & (jj < N_PART)).astype(f32)
    mcol = jnp.broadcast_to(mask4, (bs, NI, NP, 1)).reshape(R, 1)

    bf16 = jnp.bfloat16

    def dot(x, w):
        return jnp.dot(x.astype(bf16), w, preferred_element_type=f32)

    def rnd(x):
        return x.astype(bf16).astype(f32)

    def pair_diff(co):
        return (co[:, :NI, None, :] - co[:, None, :, :]).reshape(R, S_DIM)

    cd0 = pair_diff(coord)
    ea_col = jnp.sum(cd0 * cd0, axis=-1, keepdims=True)

    zpad3 = jnp.zeros((bs, NP - NI, S_DIM), f32)
    zpadh = jnp.zeros((bs, NP - NI, HID), f32)

    RH = R // 2
    NH = (bs // 2) * NP
    bsh = bs // 2
    mh = mcol[:RH]
    H2 = 2 * HID

    def halves(x):
        return x[:, :HID], x[:, HID:]

    def cat_bcast(u, v):
        return jnp.concatenate(
            [jnp.broadcast_to(u, (u.shape[0], HID)),
             jnp.broadcast_to(v, (v.shape[0], HID))], axis=1)

    def _lnk2(x, g, b):
        x1, x2 = halves(x)
        xc = x - cat_bcast(jnp.mean(x1, axis=-1, keepdims=True),
                           jnp.mean(x2, axis=-1, keepdims=True))
        s1, s2 = halves(xc * xc)
        den = cat_bcast(
            jnp.sqrt(jnp.mean(s1, axis=-1, keepdims=True) + 1e-5),
            jnp.sqrt(jnp.mean(s2, axis=-1, keepdims=True) + 1e-5))
        return xc / den * g + b

    for l in range(N_LAYERS):
        cd = pair_diff(coord)
        rad_col = jnp.sum(cd * cd, axis=-1, keepdims=True)
        cdn = cd / (jnp.sqrt(rad_col + 1e-8) + 1.0)

        ac = dot(h, eW1ab_ref[l])
        a = ac[:, :HID] + eb1_ref[l]
        c = ac[:, HID:]
        ap = jnp.concatenate([a[:NH], a[NH:]], axis=1)
        cp = jnp.concatenate([c[:NH], c[NH:]], axis=1)
        a4 = ap.reshape(bsh, NP, 1, H2)[:, :NI]
        c4 = cp.reshape(bsh, 1, NP, H2)
        radcat = jnp.concatenate(
            [rad_col[:RH], ea_col[:RH], rad_col[RH:], ea_col[RH:]], axis=1)
        pre = (a4 + c4).reshape(RH, H2) + dot(radcat, wrwe_ref[l])
        m = _silu(_lnk2(pre, eg1_ref[l], ebe1_ref[l]))
        m = _silu(_lnk2(dot(m, eW2_ref[l]) + eb2_ref[l], eg2_ref[l],
                        ebe2_ref[l]))
        s1, s2 = halves(rnd(m) * aw_ref[l])
        att = cat_bcast(
            jax.nn.sigmoid(jnp.sum(s1, axis=-1, keepdims=True)
                           + ab_ref[l]) * mh,
            jax.nn.sigmoid(jnp.sum(s2, axis=-1, keepdims=True)
                           + ab_ref[l]) * mh)
        ef = m * att

        c1 = _silu(_lnk2(dot(ef, cW1_ref[l]) + cb1_ref[l], cg1_ref[l],
                         cbe1_ref[l]))
        t1, t2 = halves(rnd(c1) * cw2_ref[l])
        c2 = jnp.concatenate(
            [jnp.sum(t1, axis=-1, keepdims=True) * mh,
             jnp.sum(t2, axis=-1, keepdims=True) * mh], axis=0)

        dlt = jnp.sum((cdn * c2).reshape(bs, NI, NP, S_DIM), axis=2)
        coord = coord + jnp.concatenate([dlt, zpad3], axis=1)

        aggp = jnp.sum(ef.reshape(bsh, NI, NP, H2), axis=2)
        aggp = jnp.concatenate(
            [aggp, jnp.zeros((bsh, NP - NI, H2), f32)],
            axis=1).reshape(NH, H2)
        agg = jnp.concatenate([aggp[:, :HID], aggp[:, HID:]], axis=0)
        hagg = jnp.concatenate([h, agg], axis=-1)
        o = _silu(_lnk(dot(hagg, nW1_ref[l]) + nb1_ref[l],
                       ng1_ref[l], nbe1_ref[l]))
        h = h + dot(o, nW2_ref[l]) + nb2_ref[l]

    nmask = (lax.broadcasted_iota(jnp.int32, (1, NP, 1), 1)
             < N_PART).astype(f32)
    vel = (coord - x_init) * nmask
    mean = jnp.sum(vel, axis=1, keepdims=True) * (1.0 / N_PART)
    out_ref[...] = (vel - mean) * nmask


@jax.jit
def kernel(x_flat, params):
    B = x_flat.shape[0]
    coord0 = x_flat.reshape(B, N_PART, S_DIM)
    coord0 = jnp.pad(coord0, ((0, 0), (0, NP - N_PART), (0, 0)))
    bf16 = jnp.bfloat16

    def rnd(x):
        return x.astype(bf16).astype(jnp.float32)

    h0 = (rnd(params['emb_W'][:, 0]) + params['emb_b'])[None, :]

    L = params['layers']

    def st(f):
        return jnp.stack([f(p) for p in L])

    def bd(w):
        z = jnp.zeros_like(w)
        return jnp.block([[w, z], [z, w]])

    def t2(v):
        return jnp.concatenate([v, v], axis=-1)

    zv = jnp.zeros((HID,), jnp.float32)

    ops = [
        coord0,
        h0,
        st(lambda p: jnp.concatenate(
            [p['e_W1'][:, :HID].T, p['e_W1'][:, HID:2 * HID].T],
            axis=-1)).astype(bf16),
        st(lambda p: jnp.stack(
            [jnp.concatenate([p['e_W1'][:, 2 * HID], zv]),
             jnp.concatenate([p['e_W1'][:, 2 * HID + 1], zv]),
             jnp.concatenate([zv, p['e_W1'][:, 2 * HID]]),
             jnp.concatenate([zv, p['e_W1'][:, 2 * HID + 1]])],
            axis=0)).astype(bf16),
        st(lambda p: p['e_b1'][None, :]),
        st(lambda p: t2(p['e_g1'][None, :])),
        st(lambda p: t2(p['e_be1'][None, :])),
        st(lambda p: bd(p['e_W2'].T)).astype(bf16),
        st(lambda p: t2(p['e_b2'][None, :])),
        st(lambda p: t2(p['e_g2'][None, :])),
        st(lambda p: t2(p['e_be2'][None, :])),
        st(lambda p: t2(rnd(p['a_W'][0][None, :]))),
        st(lambda p: p['a_b'][None, :]),
        st(lambda p: bd(p['c_W1'].T)).astype(bf16),
        st(lambda p: t2(p['c_b1'][None, :])),
        st(lambda p: t2(p['c_g1'][None, :])),
        st(lambda p: t2(p['c_be1'][None, :])),
        st(lambda p: t2(rnd(p['c_W2'][0][None, :]))),
        st(lambda p: p['n_W1'].T).astype(bf16),
        st(lambda p: p['n_b1'][None, :]),
        st(lambda p: p['n_g1'][None, :]),
        st(lambda p: p['n_be1'][None, :]),
        st(lambda p: p['n_W2'].T).astype(bf16),
        st(lambda p: p['n_b2'][None, :]),
    ]

    in_specs = [pl.BlockSpec((BS, NP, S_DIM), lambda i: (i, 0, 0))]
    for op in ops[1:]:
        shp = op.shape
        in_specs.append(
            pl.BlockSpec(shp, functools.partial(
                lambda nd, i: (0,) * nd, len(shp))))

    out = pl.pallas_call(
        _egnn_kernel,
        grid=(B // BS,),
        in_specs=in_specs,
        out_specs=pl.BlockSpec((BS, NP, S_DIM), lambda i: (i, 0, 0)),
        out_shape=jax.ShapeDtypeStruct((B, NP, S_DIM), jnp.float32),
        compiler_params=pltpu.CompilerParams(
            dimension_semantics=("parallel",)),
    )(*ops)

    return out[:, :N_PART, :].reshape(B, N_PART * S_DIM)


# speedup vs baseline: 1.1943x; 1.1943x over previous
import functools

import jax
import jax.numpy as jnp
from jax import lax
from jax.experimental import pallas as pl
from jax.experimental.pallas import tpu as pltpu

N_PART = 13
NP = 16
S_DIM = 3
HID = 64
N_LAYERS = 5
BS = 64


def _lnk(x, g, b):
    m = jnp.mean(x, axis=-1, keepdims=True)
    v = jnp.mean((x - m) ** 2, axis=-1, keepdims=True)
    return (x - m) / jnp.sqrt(v + 1e-5) * g + b


def _silu(x):
    return x * jax.nn.sigmoid(x)


def _egnn_kernel(coord_ref, h0_ref, eW1ab_ref, wrwe_ref,
                 eb1_ref, eg1_ref, ebe1_ref, eW2_ref, eb2_ref, eg2_ref,
                 ebe2_ref, aw_ref, ab_ref, cW1_ref, cb1_ref, cg1_ref,
                 cbe1_ref, cw2_ref, nW1_ref, nb1_ref, ng1_ref,
                 nbe1_ref, nW2_ref, nb2_ref, out_ref):
    bs = coord_ref.shape[0]
    NI = N_PART
    R = bs * NI * NP
    f32 = jnp.float32

    coord = coord_ref[...]
    x_init = coord
    h = jnp.broadcast_to(h0_ref[...], (bs * NP, HID))

    ii = lax.broadcasted_iota(jnp.int32, (1, NI, NP, 1), 1)
    jj = lax.broadcasted_iota(jnp.int32, (1, NI, NP, 1), 2)
    mask4 = ((ii != jj) & (jj < N_PART)).astype(f32)
    mcol = jnp.broadcast_to(mask4, (bs, NI, NP, 1)).reshape(R, 1)

    bf16 = jnp.bfloat16

    def dot(x, w):
        return jnp.dot(x.astype(bf16), w, preferred_element_type=f32)

    def rnd(x):
        return x.astype(bf16).astype(f32)

    def pair_diff(co):
        return (co[:, :NI, None, :] - co[:, None, :, :]).reshape(R, S_DIM)

    cd0 = pair_diff(coord)
    ea_col = jnp.sum(cd0 * cd0, axis=-1, keepdims=True)

    zpad3 = jnp.zeros((bs, NP - NI, S_DIM), f32)
    zpadh = jnp.zeros((bs, NP - NI, HID), f32)

    for l in range(N_LAYERS):
        cd = pair_diff(coord)
        rad_col = jnp.sum(cd * cd, axis=-1, keepdims=True)
        cdn = cd / (jnp.sqrt(rad_col + 1e-8) + 1.0)

        ac = dot(h, eW1ab_ref[l])
        a = (ac[:, :HID] + eb1_ref[l]).reshape(bs, NP, 1, HID)[:, :NI]
        c = ac[:, HID:].reshape(bs, 1, NP, HID)
        pre = ((a + c).reshape(R, HID)
               + dot(jnp.concatenate([rad_col, ea_col], axis=1),
                     wrwe_ref[l]))
        m = _silu(_lnk(pre, eg1_ref[l], ebe1_ref[l]))
        m = _silu(_lnk(dot(m, eW2_ref[l]) + eb2_ref[l], eg2_ref[l],
                       ebe2_ref[l]))
        att = jax.nn.sigmoid(dot(m, aw_ref[l]) + ab_ref[l])
        ef = m * (att * mcol)

        c1 = _silu(_lnk(dot(ef, cW1_ref[l]) + cb1_ref[l], cg1_ref[l],
                        cbe1_ref[l]))
        c2 = dot(c1, cw2_ref[l]) * mcol

        dlt = jnp.sum((cdn * c2).reshape(bs, NI, NP, S_DIM), axis=2)
        coord = coord + jnp.concatenate([dlt, zpad3], axis=1)

        agg = jnp.sum(ef.reshape(bs, NI, NP, HID), axis=2)
        agg = jnp.concatenate([agg, zpadh], axis=1).reshape(bs * NP, HID)
        hagg = jnp.concatenate([h, agg], axis=-1)
        o = _silu(_lnk(dot(hagg, nW1_ref[l]) + nb1_ref[l],
                       ng1_ref[l], nbe1_ref[l]))
        h = h + dot(o, nW2_ref[l]) + nb2_ref[l]

    nmask = (lax.broadcasted_iota(jnp.int32, (1, NP, 1), 1)
             < N_PART).astype(f32)
    vel = (coord - x_init) * nmask
    mean = jnp.sum(vel, axis=1, keepdims=True) * (1.0 / N_PART)
    out_ref[...] = (vel - mean) * nmask


@jax.jit
def kernel(x_flat, params):
    B = x_flat.shape[0]
    coord0 = x_flat.reshape(B, N_PART, S_DIM)
    coord0 = jnp.pad(coord0, ((0, 0), (0, NP - N_PART), (0, 0)))
    bf16 = jnp.bfloat16

    def rnd(x):
        return x.astype(bf16).astype(jnp.float32)

    h0 = (rnd(params['emb_W'][:, 0]) + params['emb_b'])[None, :]

    L = params['layers']

    def st(f):
        return jnp.stack([f(p) for p in L])

    ops = [
        coord0,
        h0,
        st(lambda p: jnp.concatenate(
            [p['e_W1'][:, :HID].T, p['e_W1'][:, HID:2 * HID].T],
            axis=-1)).astype(bf16),
        st(lambda p: jnp.stack(
            [p['e_W1'][:, 2 * HID], p['e_W1'][:, 2 * HID + 1]],
            axis=0)).astype(bf16),
        st(lambda p: p['e_b1'][None, :]),
        st(lambda p: p['e_g1'][None, :]),
        st(lambda p: p['e_be1'][None, :]),
        st(lambda p: p['e_W2'].T).astype(bf16),
        st(lambda p: p['e_b2'][None, :]),
        st(lambda p: p['e_g2'][None, :]),
        st(lambda p: p['e_be2'][None, :]),
        st(lambda p: p['a_W'].T).astype(bf16),
        st(lambda p: p['a_b'][None, :]),
        st(lambda p: p['c_W1'].T).astype(bf16),
        st(lambda p: p['c_b1'][None, :]),
        st(lambda p: p['c_g1'][None, :]),
        st(lambda p: p['c_be1'][None, :]),
        st(lambda p: p['c_W2'].T).astype(bf16),
        st(lambda p: p['n_W1'].T).astype(bf16),
        st(lambda p: p['n_b1'][None, :]),
        st(lambda p: p['n_g1'][None, :]),
        st(lambda p: p['n_be1'][None, :]),
        st(lambda p: p['n_W2'].T).astype(bf16),
        st(lambda p: p['n_b2'][None, :]),
    ]

    in_specs = [pl.BlockSpec((BS, NP, S_DIM), lambda i: (i, 0, 0))]
    for op in ops[1:]:
        shp = op.shape
        in_specs.append(
            pl.BlockSpec(shp, functools.partial(
                lambda nd, i: (0,) * nd, len(shp))))

    out = pl.pallas_call(
        _egnn_kernel,
        grid=(B // BS,),
        in_specs=in_specs,
        out_specs=pl.BlockSpec((BS, NP, S_DIM), lambda i: (i, 0, 0)),
        out_shape=jax.ShapeDtypeStruct((B, NP, S_DIM), jnp.float32),
        compiler_params=pltpu.CompilerParams(
            dimension_semantics=("parallel",)),
    )(*ops)

    return out[:, :N_PART, :].reshape(B, N_PART * S_DIM)
